# baseline (speedup 1.0000x reference)
import functools

import jax
import jax.numpy as jnp
from jax import lax
from jax.experimental import pallas as pl
from jax.experimental.pallas import tpu as pltpu
from jax.experimental.pallas import tpu_sc as plsc

N = 10000
E = 320000
D = 128

NC = 2
NS = 16
NW = NC * NS
L = 16

CHUNK = 128
CPT0 = 144
CPT1 = 16
NPART = 2 if CPT1 else 1
E_PAD = NS * (CPT0 + CPT1) * CHUNK
PAD_EDGES = E_PAD - E

N_ACC = 10240
HR = 80

_f32 = jnp.float32
_i32 = jnp.int32


def _zero16():
    return jnp.zeros((L,), _f32)


def _mesh():
    return plsc.VectorSubcoreMesh(
        core_axis_name="c", subcore_axis_name="s", num_cores=NC, num_subcores=NS)


IB = 8


def _tile_layout(cid, sid):
    base = jnp.where(cid == 0, sid * CPT0, NS * CPT0 + sid * CPT1)
    nb = jnp.where(cid == 0, CPT0 // IB, CPT1 // IB)
    return base, nb


def _sc_hist_build():
    scratch = [
        pltpu.VMEM((IB, CHUNK), _i32),
        pltpu.VMEM((IB, CHUNK), _i32),
        pltpu.VMEM((HR, D), _f32),
        pltpu.VMEM((HR, D), _f32),
        pltpu.VMEM((HR,), _i32),
        pltpu.VMEM_SHARED((HR, D), _f32),
        pltpu.VMEM_SHARED((HR, D), _f32),
    ]

    def body(srcb, dstb, hs_out, hd_out,
             sidx, didx, hsrc_v, hdst_v, rid_v, hs_sh, hd_sh):
        cid = lax.axis_index("c")
        sid = lax.axis_index("s")
        base, nb = _tile_layout(cid, sid)

        def zhist(i, c):
            hsrc_v[i >> 3, pl.ds((i & 7) * L, L)] = _zero16()
            hdst_v[i >> 3, pl.ds((i & 7) * L, L)] = _zero16()
            return c
        lax.fori_loop(0, HR * (D // L), zhist, 0)
        iota16 = lax.iota(_i32, L)
        for k in range(HR // L):
            rid_v[pl.ds(k * L, L)] = iota16 + k * L

        @pl.when(sid == 0)
        def _():
            pltpu.sync_copy(hsrc_v, hs_sh)
            pltpu.sync_copy(hdst_v, hd_sh)
        plsc.subcore_barrier()

        ones16 = jnp.ones((L,), _f32)

        def outer(ob, c):
            pltpu.sync_copy(srcb.at[pl.ds(base + ob * IB, IB)], sidx)
            pltpu.sync_copy(dstb.at[pl.ds(base + ob * IB, IB)], didx)
            for j in range(IB):
                for v in range(CHUNK // L):
                    s16 = sidx[j, pl.ds(v * L, L)]
                    d16 = didx[j, pl.ds(v * L, L)]
                    plsc.addupdate_scatter(hsrc_v, [s16 >> 7, s16 & 127], ones16)
                    plsc.addupdate_scatter(hdst_v, [d16 >> 7, d16 & 127], ones16)
            return c
        lax.fori_loop(0, nb, outer, 0)

        pltpu.sync_copy(hsrc_v, hs_sh.at[rid_v], add=True)
        pltpu.sync_copy(hdst_v, hd_sh.at[rid_v], add=True)
        plsc.subcore_barrier()

        @pl.when(sid == 0)
        def _():
            pltpu.sync_copy(hs_sh, hs_out.at[cid])
            pltpu.sync_copy(hd_sh, hd_out.at[cid])

    return pl.kernel(
        body,
        out_type=[jax.ShapeDtypeStruct((NC, HR, D), _f32)] * 2,
        mesh=_mesh(), scratch_types=scratch,
        compiler_params=pltpu.CompilerParams(needs_layout_passes=False))


def _sc_agg_build():
    scratch = [
        pltpu.VMEM((IB, CHUNK), _i32),
        pltpu.VMEM((IB, CHUNK), _i32),
        pltpu.VMEM((IB, CHUNK), _i32),
        pltpu.VMEM((IB, CHUNK), _i32),
        pltpu.VMEM((CHUNK, D), _f32),
        pltpu.VMEM((CHUNK, D), _f32),
        pltpu.VMEM_SHARED((N_ACC, D), _f32),
        pltpu.SemaphoreType.DMA,
        pltpu.SemaphoreType.DMA,
        pltpu.SemaphoreType.DMA,
        pltpu.SemaphoreType.DMA,
        pltpu.SemaphoreType.DMA,
        pltpu.SemaphoreType.DMA,
    ]

    def body(table, srcb, dstb, part,
             s0, d0, s1, d1, bufA, bufB, acc_sh,
             sgA, sgB, ssA, ssB, si0, si1):
        cid = lax.axis_index("c")
        sid = lax.axis_index("s")
        base, nb = _tile_layout(cid, sid)
        nbp = nb // 2
        SETS = [(s0, d0, si0), (s1, d1, si1)]
        BUFS = [(bufA, sgA, ssA), (bufB, sgB, ssB)]

        def idx_start(b, blk):
            s_, d_, si_ = SETS[b]
            pltpu.async_copy(srcb.at[pl.ds(base + blk * IB, IB)], s_, si_)
            pltpu.async_copy(dstb.at[pl.ds(base + blk * IB, IB)], d_, si_)

        def idx_wait(b, blk):
            s_, d_, si_ = SETS[b]
            pltpu.make_async_copy(srcb.at[pl.ds(base + blk * IB, IB)], s_, si_).wait()
            pltpu.make_async_copy(dstb.at[pl.ds(base + blk * IB, IB)], d_, si_).wait()

        def g_start(bb, sb, j):
            s_, _, _ = SETS[sb]
            buf, sg, _ = BUFS[bb]
            pltpu.async_copy(table.at[s_.at[j]], buf, sg)

        def g_wait(bb, sb, j):
            s_, _, _ = SETS[sb]
            buf, sg, _ = BUFS[bb]
            pltpu.make_async_copy(table.at[s_.at[j]], buf, sg).wait()

        def s_start(bb, sb, j):
            _, d_, _ = SETS[sb]
            buf, _, ss = BUFS[bb]
            pltpu.async_copy(buf, acc_sh.at[d_.at[j]], ss, add=True)

        def s_wait(bb, sb, j):
            _, d_, _ = SETS[sb]
            buf, _, ss = BUFS[bb]
            pltpu.make_async_copy(buf, acc_sh.at[d_.at[j]], ss).wait()

        @pl.when(cid < NPART)
        def _core_body():
            def zrow(i, c):
                bufA[i >> 3, pl.ds((i & 7) * L, L)] = _zero16()
                return c
            lax.fori_loop(0, CHUNK * (D // L), zrow, 0)
            for k in range(N_ACC // NS // CHUNK):
                pltpu.sync_copy(bufA, acc_sh.at[pl.ds(sid * (N_ACC // NS) + k * CHUNK, CHUNK)])
            plsc.subcore_barrier()

            idx_start(0, 0)
            idx_wait(0, 0)
            idx_start(1, 1)
            g_start(0, 0, 0)

            def outer(p, c):
                for half in range(2):
                    sb = half
                    for j in range(IB):
                        bb = j % 2
                        ob = (j + 1) % 2
                        g_wait(bb, sb, j)
                        if half == 0 and j == 0:
                            @pl.when(p > 0)
                            def _():
                                s_wait(1, 1, IB - 1)
                        elif half == 1 and j == 0:
                            s_wait(1, 0, IB - 1)
                        else:
                            s_wait(ob, sb, j - 1)
                        if j == 0:
                            if half == 0:
                                @pl.when(p > 0)
                                def _():
                                    idx_start(1, 2 * p + 1)
                            else:
                                @pl.when(p < nbp - 1)
                                def _():
                                    idx_start(0, 2 * p + 2)
                        if j < IB - 1:
                            g_start(ob, sb, j + 1)
                        elif half == 0:
                            idx_wait(1, 2 * p + 1)
                            g_start(0, 1, 0)
                        else:
                            @pl.when(p < nbp - 1)
                            def _():
                                idx_wait(0, 2 * p + 2)
                                g_start(0, 0, 0)
                        s_start(bb, sb, j)
                return c
            lax.fori_loop(0, nbp, outer, 0)
            s_wait(1, 1, IB - 1)
            plsc.subcore_barrier()

            RPT = N_ACC // NS
            r0 = sid * RPT
            dst = part.at[cid, pl.ds(r0, RPT)] if NPART == 2 else \
                part.at[pl.ds(r0, RPT)]
            pltpu.sync_copy(acc_sh.at[pl.ds(r0, RPT)], dst)

    out_shape = (NPART, N_ACC, D) if NPART == 2 else (N_ACC, D)
    return pl.kernel(
        body,
        out_type=[jax.ShapeDtypeStruct(out_shape, _f32)],
        mesh=_mesh(), scratch_types=scratch,
        compiler_params=pltpu.CompilerParams(needs_layout_passes=False))


@functools.lru_cache(maxsize=None)
def _sc_hist_cached():
    return _sc_hist_build()


@functools.lru_cache(maxsize=None)
def _sc_agg_cached():
    return _sc_agg_build()


def _sc_hist(srcb, dstb):
    return _sc_hist_cached()(srcb, dstb)


def _sc_agg_run(table, srcb, dstb):
    return _sc_agg_cached()(table, srcb, dstb)[0]


def _psum(p_ref):
    if NPART == 2:
        return p_ref[0, :N, :] + p_ref[1, :N, :]
    return p_ref[:N, :]


def _tc_mm(x, W):
    def body(x_ref, w_ref, o_ref):
        o_ref[...] = jnp.dot(x_ref[...], w_ref[...], preferred_element_type=_f32)
    return pl.pallas_call(
        body, out_shape=jax.ShapeDtypeStruct((N, D), _f32))(x, W)


def _tc_sage_combine(part, y, degi, b, W):
    def body(p_ref, y_ref, d_ref, b_ref, w_ref, o_ref):
        deg = d_ref[0] + d_ref[1]
        h = (_psum(p_ref) + y_ref[...]) / (deg + 1.0) + b_ref[...][None, :]
        o_ref[...] = jnp.dot(h, w_ref[...], preferred_element_type=_f32)
    return pl.pallas_call(
        body, out_shape=jax.ShapeDtypeStruct((N, D), _f32))(part, y, degi, b, W)


def _tc_sage_combine_scale(part, y, degi, dego, b, W):
    def body(p_ref, y_ref, di_ref, do_ref, b_ref, w_ref, o_ref):
        degi = di_ref[0] + di_ref[1]
        row0 = (lax.broadcasted_iota(_i32, (N, 1), 0) == 0).astype(_f32)
        dego = do_ref[0] + do_ref[1] - PAD_EDGES * row0
        h = (_psum(p_ref) + y_ref[...]) / (degi + 1.0) + b_ref[...][None, :]
        y3 = jnp.dot(h, w_ref[...], preferred_element_type=_f32)
        o_ref[...] = y3 * lax.rsqrt(jnp.maximum(dego, 1.0))
    return pl.pallas_call(
        body, out_shape=jax.ShapeDtypeStruct((N, D), _f32))(part, y, degi, dego, b, W)


def _tc_final(part, degi, b):
    def body(p_ref, d_ref, b_ref, o_ref):
        deg = d_ref[0] + d_ref[1]
        o_ref[...] = (_psum(p_ref) * lax.rsqrt(jnp.maximum(deg, 1.0))
                      + b_ref[...][None, :])
    return pl.pallas_call(
        body, out_shape=jax.ShapeDtypeStruct((N, D), _f32))(part, degi, b)


def kernel(x, edge_index, W1, b1, W2, b2, W3, b3):
    src = edge_index[0]
    dst = edge_index[1]
    pad_dst = N + (jnp.arange(PAD_EDGES, dtype=_i32) % (N_ACC - N))
    src_p = jnp.concatenate([src, jnp.zeros((PAD_EDGES,), _i32)]).reshape(E_PAD // CHUNK, CHUNK)
    dst_p = jnp.concatenate([dst, pad_dst]).reshape(E_PAD // CHUNK, CHUNK)

    hs, hd = _sc_hist(src_p, dst_p)
    y1 = _tc_mm(x, W1)
    p1 = _sc_agg_run(y1, src_p, dst_p)

    degi = hd.reshape(NC, N_ACC)[:, :N, None]
    dego = hs.reshape(NC, N_ACC)[:, :N, None]

    y2 = _tc_sage_combine(p1, y1, degi, b1, W2)
    p2 = _sc_agg_run(y2, src_p, dst_p)
    y3 = _tc_sage_combine_scale(p2, y2, degi, dego, b2, W3)
    p3 = _sc_agg_run(y3, src_p, dst_p)
    return _tc_final(p3, degi, b3)


# speedup vs baseline: 1.5326x; 1.5326x over previous
import functools

import jax
import jax.numpy as jnp
from jax import lax
from jax.experimental import pallas as pl
from jax.experimental.pallas import tpu as pltpu
from jax.experimental.pallas import tpu_sc as plsc

N = 10000
E = 320000
D = 128

NC = 2
NS = 16
NW = NC * NS
L = 16

CHUNK = 128
CPT0 = 160
CPT1 = 0
NPART = 2 if CPT1 else 1
E_PAD = NS * (CPT0 + CPT1) * CHUNK
PAD_EDGES = E_PAD - E

N_ACC = 10240
HR = 80

_f32 = jnp.float32
_i32 = jnp.int32


def _zero16():
    return jnp.zeros((L,), _f32)


def _mesh():
    return plsc.VectorSubcoreMesh(
        core_axis_name="c", subcore_axis_name="s", num_cores=NC, num_subcores=NS)


IB = 8


def _tile_layout(cid, sid):
    base = jnp.where(cid == 0, sid * CPT0, NS * CPT0 + sid * CPT1)
    nb = jnp.where(cid == 0, CPT0 // IB, CPT1 // IB)
    return base, nb


def _sc_hist_build():
    scratch = [
        pltpu.VMEM((IB, CHUNK), _i32),
        pltpu.VMEM((IB, CHUNK), _i32),
        pltpu.VMEM((HR, D), _f32),
        pltpu.VMEM((HR, D), _f32),
        pltpu.VMEM((HR,), _i32),
        pltpu.VMEM_SHARED((HR, D), _f32),
        pltpu.VMEM_SHARED((HR, D), _f32),
    ]

    def body(srcb, dstb, hs_out, hd_out,
             sidx, didx, hsrc_v, hdst_v, rid_v, hs_sh, hd_sh):
        cid = lax.axis_index("c")
        sid = lax.axis_index("s")
        base, nb = _tile_layout(cid, sid)

        def zhist(i, c):
            hsrc_v[i >> 3, pl.ds((i & 7) * L, L)] = _zero16()
            hdst_v[i >> 3, pl.ds((i & 7) * L, L)] = _zero16()
            return c
        lax.fori_loop(0, HR * (D // L), zhist, 0)
        iota16 = lax.iota(_i32, L)
        for k in range(HR // L):
            rid_v[pl.ds(k * L, L)] = iota16 + k * L

        @pl.when(sid == 0)
        def _():
            pltpu.sync_copy(hsrc_v, hs_sh)
            pltpu.sync_copy(hdst_v, hd_sh)
        plsc.subcore_barrier()

        ones16 = jnp.ones((L,), _f32)

        def outer(ob, c):
            pltpu.sync_copy(srcb.at[pl.ds(base + ob * IB, IB)], sidx)
            pltpu.sync_copy(dstb.at[pl.ds(base + ob * IB, IB)], didx)
            for j in range(IB):
                for v in range(CHUNK // L):
                    s16 = sidx[j, pl.ds(v * L, L)]
                    d16 = didx[j, pl.ds(v * L, L)]
                    plsc.addupdate_scatter(hsrc_v, [s16 >> 7, s16 & 127], ones16)
                    plsc.addupdate_scatter(hdst_v, [d16 >> 7, d16 & 127], ones16)
            return c
        lax.fori_loop(0, nb, outer, 0)

        pltpu.sync_copy(hsrc_v, hs_sh.at[rid_v], add=True)
        pltpu.sync_copy(hdst_v, hd_sh.at[rid_v], add=True)
        plsc.subcore_barrier()

        @pl.when(sid == 0)
        def _():
            pltpu.sync_copy(hs_sh, hs_out.at[cid])
            pltpu.sync_copy(hd_sh, hd_out.at[cid])

    return pl.kernel(
        body,
        out_type=[jax.ShapeDtypeStruct((NC, HR, D), _f32)] * 2,
        mesh=_mesh(), scratch_types=scratch,
        compiler_params=pltpu.CompilerParams(needs_layout_passes=False))


def _sc_agg_build():
    scratch = [
        pltpu.VMEM((IB, CHUNK), _i32),
        pltpu.VMEM((IB, CHUNK), _i32),
        pltpu.VMEM((IB, CHUNK), _i32),
        pltpu.VMEM((IB, CHUNK), _i32),
        pltpu.VMEM((CHUNK, D), _f32),
        pltpu.VMEM((CHUNK, D), _f32),
        pltpu.VMEM_SHARED((N_ACC, D), _f32),
        pltpu.SemaphoreType.DMA,
        pltpu.SemaphoreType.DMA,
        pltpu.SemaphoreType.DMA,
        pltpu.SemaphoreType.DMA,
        pltpu.SemaphoreType.DMA,
        pltpu.SemaphoreType.DMA,
    ]

    def body(table, srcb, dstb, part,
             s0, d0, s1, d1, bufA, bufB, acc_sh,
             sgA, sgB, ssA, ssB, si0, si1):
        cid = lax.axis_index("c")
        sid = lax.axis_index("s")
        base, nb = _tile_layout(cid, sid)
        nbp = nb // 2
        SETS = [(s0, d0, si0), (s1, d1, si1)]
        BUFS = [(bufA, sgA, ssA), (bufB, sgB, ssB)]

        def idx_start(b, blk):
            s_, d_, si_ = SETS[b]
            pltpu.async_copy(srcb.at[pl.ds(base + blk * IB, IB)], s_, si_)
            pltpu.async_copy(dstb.at[pl.ds(base + blk * IB, IB)], d_, si_)

        def idx_wait(b, blk):
            s_, d_, si_ = SETS[b]
            pltpu.make_async_copy(srcb.at[pl.ds(base + blk * IB, IB)], s_, si_).wait()
            pltpu.make_async_copy(dstb.at[pl.ds(base + blk * IB, IB)], d_, si_).wait()

        def g_start(bb, sb, j):
            s_, _, _ = SETS[sb]
            buf, sg, _ = BUFS[bb]
            pltpu.async_copy(table.at[s_.at[j]], buf, sg)

        def g_wait(bb, sb, j):
            s_, _, _ = SETS[sb]
            buf, sg, _ = BUFS[bb]
            pltpu.make_async_copy(table.at[s_.at[j]], buf, sg).wait()

        def s_start(bb, sb, j):
            _, d_, _ = SETS[sb]
            buf, _, ss = BUFS[bb]
            pltpu.async_copy(buf, acc_sh.at[d_.at[j]], ss, add=True)

        def s_wait(bb, sb, j):
            _, d_, _ = SETS[sb]
            buf, _, ss = BUFS[bb]
            pltpu.make_async_copy(buf, acc_sh.at[d_.at[j]], ss).wait()

        @pl.when(cid < NPART)
        def _core_body():
            def zrow(i, c):
                bufA[i >> 3, pl.ds((i & 7) * L, L)] = _zero16()
                return c
            lax.fori_loop(0, CHUNK * (D // L), zrow, 0)
            for k in range(N_ACC // NS // CHUNK):
                pltpu.sync_copy(bufA, acc_sh.at[pl.ds(sid * (N_ACC // NS) + k * CHUNK, CHUNK)])
            plsc.subcore_barrier()

            idx_start(0, 0)
            idx_wait(0, 0)
            idx_start(1, 1)
            g_start(0, 0, 0)

            def outer(p, c):
                for half in range(2):
                    sb = half
                    for j in range(IB):
                        bb = j % 2
                        ob = (j + 1) % 2
                        g_wait(bb, sb, j)
                        if half == 0 and j == 0:
                            @pl.when(p > 0)
                            def _():
                                s_wait(1, 1, IB - 1)
                        elif half == 1 and j == 0:
                            s_wait(1, 0, IB - 1)
                        else:
                            s_wait(ob, sb, j - 1)
                        if j == 0:
                            if half == 0:
                                @pl.when(p > 0)
                                def _():
                                    idx_start(1, 2 * p + 1)
                            else:
                                @pl.when(p < nbp - 1)
                                def _():
                                    idx_start(0, 2 * p + 2)
                        if j < IB - 1:
                            g_start(ob, sb, j + 1)
                        elif half == 0:
                            idx_wait(1, 2 * p + 1)
                            g_start(0, 1, 0)
                        else:
                            @pl.when(p < nbp - 1)
                            def _():
                                idx_wait(0, 2 * p + 2)
                                g_start(0, 0, 0)
                        s_start(bb, sb, j)
                return c
            lax.fori_loop(0, nbp, outer, 0)
            s_wait(1, 1, IB - 1)
            plsc.subcore_barrier()

            RPT = N_ACC // NS
            r0 = sid * RPT
            dst = part.at[cid, pl.ds(r0, RPT)] if NPART == 2 else \
                part.at[pl.ds(r0, RPT)]
            pltpu.sync_copy(acc_sh.at[pl.ds(r0, RPT)], dst)

    out_shape = (NPART, N_ACC, D) if NPART == 2 else (N_ACC, D)
    return pl.kernel(
        body,
        out_type=[jax.ShapeDtypeStruct(out_shape, _f32)],
        mesh=_mesh(), scratch_types=scratch,
        compiler_params=pltpu.CompilerParams(needs_layout_passes=False))


@functools.lru_cache(maxsize=None)
def _sc_hist_cached():
    return _sc_hist_build()


@functools.lru_cache(maxsize=None)
def _sc_agg_cached():
    return _sc_agg_build()


def _sc_hist(srcb, dstb):
    return _sc_hist_cached()(srcb, dstb)


def _sc_agg_run(table, srcb, dstb):
    return _sc_agg_cached()(table, srcb, dstb)[0]


def _psum(p_ref):
    if NPART == 2:
        return p_ref[0, :N, :] + p_ref[1, :N, :]
    return p_ref[:N, :]


def _tc_mm(x, W):
    def body(x_ref, w_ref, o_ref):
        o_ref[...] = jnp.dot(x_ref[...], w_ref[...], preferred_element_type=_f32)
    return pl.pallas_call(
        body, out_shape=jax.ShapeDtypeStruct((N, D), _f32))(x, W)


def _tc_sage_combine(part, y, degi, b, W):
    def body(p_ref, y_ref, d_ref, b_ref, w_ref, o_ref):
        deg = d_ref[0] + d_ref[1]
        h = (_psum(p_ref) + y_ref[...]) / (deg + 1.0) + b_ref[...][None, :]
        o_ref[...] = jnp.dot(h, w_ref[...], preferred_element_type=_f32)
    return pl.pallas_call(
        body, out_shape=jax.ShapeDtypeStruct((N, D), _f32))(part, y, degi, b, W)


def _tc_sage_combine_scale(part, y, degi, dego, b, W):
    def body(p_ref, y_ref, di_ref, do_ref, b_ref, w_ref, o_ref):
        degi = di_ref[0] + di_ref[1]
        padc = (lax.broadcasted_iota(_i32, (N, 1), 0) < PAD_EDGES).astype(_f32)
        dego = do_ref[0] + do_ref[1] - padc
        h = (_psum(p_ref) + y_ref[...]) / (degi + 1.0) + b_ref[...][None, :]
        y3 = jnp.dot(h, w_ref[...], preferred_element_type=_f32)
        o_ref[...] = y3 * lax.rsqrt(jnp.maximum(dego, 1.0))
    return pl.pallas_call(
        body, out_shape=jax.ShapeDtypeStruct((N, D), _f32))(part, y, degi, dego, b, W)


def _tc_final(part, degi, b):
    def body(p_ref, d_ref, b_ref, o_ref):
        deg = d_ref[0] + d_ref[1]
        o_ref[...] = (_psum(p_ref) * lax.rsqrt(jnp.maximum(deg, 1.0))
                      + b_ref[...][None, :])
    return pl.pallas_call(
        body, out_shape=jax.ShapeDtypeStruct((N, D), _f32))(part, degi, b)


def kernel(x, edge_index, W1, b1, W2, b2, W3, b3):
    src = edge_index[0]
    dst = edge_index[1]
    pad_src = jnp.arange(PAD_EDGES, dtype=_i32) % N
    pad_dst = N + (jnp.arange(PAD_EDGES, dtype=_i32) % (N_ACC - N))
    src_p = jnp.concatenate([src, pad_src]).reshape(E_PAD // CHUNK, CHUNK)
    dst_p = jnp.concatenate([dst, pad_dst]).reshape(E_PAD // CHUNK, CHUNK)

    hs, hd = _sc_hist(src_p, dst_p)
    y1 = _tc_mm(x, W1)
    p1 = _sc_agg_run(y1, src_p, dst_p)

    degi = hd.reshape(NC, N_ACC)[:, :N, None]
    dego = hs.reshape(NC, N_ACC)[:, :N, None]

    y2 = _tc_sage_combine(p1, y1, degi, b1, W2)
    p2 = _sc_agg_run(y2, src_p, dst_p)
    y3 = _tc_sage_combine_scale(p2, y2, degi, dego, b2, W3)
    p3 = _sc_agg_run(y3, src_p, dst_p)
    return _tc_final(p3, degi, b3)


# speedup vs baseline: 1.5671x; 1.0225x over previous
import functools

import jax
import jax.numpy as jnp
from jax import lax
from jax.experimental import pallas as pl
from jax.experimental.pallas import tpu as pltpu
from jax.experimental.pallas import tpu_sc as plsc

N = 10000
E = 320000
D = 128

NC = 2
NS = 16
NW = NC * NS
L = 16

CHUNK = 64
CPT0 = 320
CPT1 = 0
NPART = 2 if CPT1 else 1
E_PAD = NS * (CPT0 + CPT1) * CHUNK
PAD_EDGES = E_PAD - E

N_ACC = 10240
HR = 80

_f32 = jnp.float32
_i32 = jnp.int32


def _zero16():
    return jnp.zeros((L,), _f32)


def _mesh():
    return plsc.VectorSubcoreMesh(
        core_axis_name="c", subcore_axis_name="s", num_cores=NC, num_subcores=NS)


IB = 8


def _tile_layout(cid, sid):
    base = jnp.where(cid == 0, sid * CPT0, NS * CPT0 + sid * CPT1)
    nb = jnp.where(cid == 0, CPT0 // IB, CPT1 // IB)
    return base, nb


def _sc_hist_build():
    scratch = [
        pltpu.VMEM((IB, CHUNK), _i32),
        pltpu.VMEM((IB, CHUNK), _i32),
        pltpu.VMEM((HR, D), _f32),
        pltpu.VMEM((HR, D), _f32),
        pltpu.VMEM((HR,), _i32),
        pltpu.VMEM_SHARED((HR, D), _f32),
        pltpu.VMEM_SHARED((HR, D), _f32),
    ]

    def body(srcb, dstb, hs_out, hd_out,
             sidx, didx, hsrc_v, hdst_v, rid_v, hs_sh, hd_sh):
        cid = lax.axis_index("c")
        sid = lax.axis_index("s")
        base, nb = _tile_layout(cid, sid)

        def zhist(i, c):
            hsrc_v[i >> 3, pl.ds((i & 7) * L, L)] = _zero16()
            hdst_v[i >> 3, pl.ds((i & 7) * L, L)] = _zero16()
            return c
        lax.fori_loop(0, HR * (D // L), zhist, 0)
        iota16 = lax.iota(_i32, L)
        for k in range(HR // L):
            rid_v[pl.ds(k * L, L)] = iota16 + k * L

        @pl.when(sid == 0)
        def _():
            pltpu.sync_copy(hsrc_v, hs_sh)
            pltpu.sync_copy(hdst_v, hd_sh)
        plsc.subcore_barrier()

        ones16 = jnp.ones((L,), _f32)

        def outer(ob, c):
            pltpu.sync_copy(srcb.at[pl.ds(base + ob * IB, IB)], sidx)
            pltpu.sync_copy(dstb.at[pl.ds(base + ob * IB, IB)], didx)
            for j in range(IB):
                for v in range(CHUNK // L):
                    s16 = sidx[j, pl.ds(v * L, L)]
                    d16 = didx[j, pl.ds(v * L, L)]
                    plsc.addupdate_scatter(hsrc_v, [s16 >> 7, s16 & 127], ones16)
                    plsc.addupdate_scatter(hdst_v, [d16 >> 7, d16 & 127], ones16)
            return c
        lax.fori_loop(0, nb, outer, 0)

        pltpu.sync_copy(hsrc_v, hs_sh.at[rid_v], add=True)
        pltpu.sync_copy(hdst_v, hd_sh.at[rid_v], add=True)
        plsc.subcore_barrier()

        @pl.when(sid == 0)
        def _():
            pltpu.sync_copy(hs_sh, hs_out.at[cid])
            pltpu.sync_copy(hd_sh, hd_out.at[cid])

    return pl.kernel(
        body,
        out_type=[jax.ShapeDtypeStruct((NC, HR, D), _f32)] * 2,
        mesh=_mesh(), scratch_types=scratch,
        compiler_params=pltpu.CompilerParams(needs_layout_passes=False))


def _sc_agg_build():
    NBUF = 4
    scratch = (
        [pltpu.VMEM((IB, CHUNK), _i32)] * 4 +
        [pltpu.VMEM((CHUNK, D), _f32)] * NBUF +
        [pltpu.VMEM_SHARED((N_ACC, D), _f32)] +
        [pltpu.SemaphoreType.DMA] * (2 * NBUF + 2)
    )

    def body(table, srcb, dstb, part,
             s0, d0, s1, d1, bufA, bufB, bufC, bufD, acc_sh,
             sgA, sgB, sgC, sgD, ssA, ssB, ssC, ssD, si0, si1):
        cid = lax.axis_index("c")
        sid = lax.axis_index("s")
        base, nb = _tile_layout(cid, sid)
        nbp = nb // 2
        SETS = [(s0, d0, si0), (s1, d1, si1)]
        BUFS = [(bufA, sgA, ssA), (bufB, sgB, ssB),
                (bufC, sgC, ssC), (bufD, sgD, ssD)]

        def idx_start(b, blk):
            s_, d_, si_ = SETS[b]
            pltpu.async_copy(srcb.at[pl.ds(base + blk * IB, IB)], s_, si_)
            pltpu.async_copy(dstb.at[pl.ds(base + blk * IB, IB)], d_, si_)

        def idx_wait(b, blk):
            s_, d_, si_ = SETS[b]
            pltpu.make_async_copy(srcb.at[pl.ds(base + blk * IB, IB)], s_, si_).wait()
            pltpu.make_async_copy(dstb.at[pl.ds(base + blk * IB, IB)], d_, si_).wait()

        def g_start(bb, sb, j):
            s_, _, _ = SETS[sb]
            buf, sg, _ = BUFS[bb]
            pltpu.async_copy(table.at[s_.at[j]], buf, sg)

        def g_wait(bb, sb, j):
            s_, _, _ = SETS[sb]
            buf, sg, _ = BUFS[bb]
            pltpu.make_async_copy(table.at[s_.at[j]], buf, sg).wait()

        def s_start(bb, sb, j):
            _, d_, _ = SETS[sb]
            buf, _, ss = BUFS[bb]
            pltpu.async_copy(buf, acc_sh.at[d_.at[j]], ss, add=True)

        def s_wait(bb, sb, j):
            _, d_, _ = SETS[sb]
            buf, _, ss = BUFS[bb]
            pltpu.make_async_copy(buf, acc_sh.at[d_.at[j]], ss).wait()

        @pl.when(cid < NPART)
        def _core_body():
            def zrow(i, c):
                bufA[i >> 3, pl.ds((i & 7) * L, L)] = _zero16()
                return c
            lax.fori_loop(0, CHUNK * (D // L), zrow, 0)
            for k in range(N_ACC // NS // CHUNK):
                pltpu.sync_copy(bufA, acc_sh.at[pl.ds(sid * (N_ACC // NS) + k * CHUNK, CHUNK)])
            plsc.subcore_barrier()

            idx_start(0, 0)
            idx_wait(0, 0)
            idx_start(1, 1)
            g_start(0, 0, 0)
            g_start(1, 0, 1)

            def outer(p, c):
                for half in range(2):
                    sb = half
                    ot = 1 - half
                    for j in range(IB):
                        bb = j % 4
                        g_wait(bb, sb, j)
                        if half == 0 and j <= 1:
                            @pl.when(p > 0)
                            def _():
                                s_wait((j + 2) % 4, 1, IB - 2 + j)
                        elif half == 1 and j <= 1:
                            s_wait((j + 2) % 4, 0, IB - 2 + j)
                        else:
                            s_wait((j + 2) % 4, sb, j - 2)
                        if j == 2:
                            if half == 0:
                                @pl.when(p > 0)
                                def _():
                                    idx_start(1, 2 * p + 1)
                            else:
                                @pl.when(p < nbp - 1)
                                def _():
                                    idx_start(0, 2 * p + 2)
                        if j < IB - 2:
                            g_start((j + 2) % 4, sb, j + 2)
                        elif half == 0:
                            if j == IB - 2:
                                idx_wait(1, 2 * p + 1)
                            g_start((j + 2) % 4, 1, j - (IB - 2))
                        else:
                            @pl.when(p < nbp - 1)
                            def _():
                                if j == IB - 2:
                                    idx_wait(0, 2 * p + 2)
                                g_start((j + 2) % 4, 0, j - (IB - 2))
                        s_start(bb, sb, j)
                return c
            lax.fori_loop(0, nbp, outer, 0)
            s_wait(2, 1, IB - 2)
            s_wait(3, 1, IB - 1)
            plsc.subcore_barrier()

            RPT = N_ACC // NS
            r0 = sid * RPT
            dst = part.at[cid, pl.ds(r0, RPT)] if NPART == 2 else \
                part.at[pl.ds(r0, RPT)]
            pltpu.sync_copy(acc_sh.at[pl.ds(r0, RPT)], dst)

    out_shape = (NPART, N_ACC, D) if NPART == 2 else (N_ACC, D)
    return pl.kernel(
        body,
        out_type=[jax.ShapeDtypeStruct(out_shape, _f32)],
        mesh=_mesh(), scratch_types=scratch,
        compiler_params=pltpu.CompilerParams(needs_layout_passes=False))


@functools.lru_cache(maxsize=None)
def _sc_hist_cached():
    return _sc_hist_build()


@functools.lru_cache(maxsize=None)
def _sc_agg_cached():
    return _sc_agg_build()


def _sc_hist(srcb, dstb):
    return _sc_hist_cached()(srcb, dstb)


def _sc_agg_run(table, srcb, dstb):
    return _sc_agg_cached()(table, srcb, dstb)[0]


def _psum(p_ref):
    if NPART == 2:
        return p_ref[0, :N, :] + p_ref[1, :N, :]
    return p_ref[:N, :]


def _tc_mm(x, W):
    def body(x_ref, w_ref, o_ref):
        o_ref[...] = jnp.dot(x_ref[...], w_ref[...], preferred_element_type=_f32)
    return pl.pallas_call(
        body, out_shape=jax.ShapeDtypeStruct((N, D), _f32))(x, W)


def _tc_sage_combine(part, y, degi, b, W):
    def body(p_ref, y_ref, d_ref, b_ref, w_ref, o_ref):
        deg = d_ref[0] + d_ref[1]
        h = (_psum(p_ref) + y_ref[...]) / (deg + 1.0) + b_ref[...][None, :]
        o_ref[...] = jnp.dot(h, w_ref[...], preferred_element_type=_f32)
    return pl.pallas_call(
        body, out_shape=jax.ShapeDtypeStruct((N, D), _f32))(part, y, degi, b, W)


def _tc_sage_combine_scale(part, y, degi, dego, b, W):
    def body(p_ref, y_ref, di_ref, do_ref, b_ref, w_ref, o_ref):
        degi = di_ref[0] + di_ref[1]
        padc = (lax.broadcasted_iota(_i32, (N, 1), 0) < PAD_EDGES).astype(_f32)
        dego = do_ref[0] + do_ref[1] - padc
        h = (_psum(p_ref) + y_ref[...]) / (degi + 1.0) + b_ref[...][None, :]
        y3 = jnp.dot(h, w_ref[...], preferred_element_type=_f32)
        o_ref[...] = y3 * lax.rsqrt(jnp.maximum(dego, 1.0))
    return pl.pallas_call(
        body, out_shape=jax.ShapeDtypeStruct((N, D), _f32))(part, y, degi, dego, b, W)


def _tc_final(part, degi, b):
    def body(p_ref, d_ref, b_ref, o_ref):
        deg = d_ref[0] + d_ref[1]
        o_ref[...] = (_psum(p_ref) * lax.rsqrt(jnp.maximum(deg, 1.0))
                      + b_ref[...][None, :])
    return pl.pallas_call(
        body, out_shape=jax.ShapeDtypeStruct((N, D), _f32))(part, degi, b)


def kernel(x, edge_index, W1, b1, W2, b2, W3, b3):
    src = edge_index[0]
    dst = edge_index[1]
    pad_src = jnp.arange(PAD_EDGES, dtype=_i32) % N
    pad_dst = N + (jnp.arange(PAD_EDGES, dtype=_i32) % (N_ACC - N))
    src_p = jnp.concatenate([src, pad_src]).reshape(E_PAD // CHUNK, CHUNK)
    dst_p = jnp.concatenate([dst, pad_dst]).reshape(E_PAD // CHUNK, CHUNK)

    hs, hd = _sc_hist(src_p, dst_p)
    y1 = _tc_mm(x, W1)
    p1 = _sc_agg_run(y1, src_p, dst_p)

    degi = hd.reshape(NC, N_ACC)[:, :N, None]
    dego = hs.reshape(NC, N_ACC)[:, :N, None]

    y2 = _tc_sage_combine(p1, y1, degi, b1, W2)
    p2 = _sc_agg_run(y2, src_p, dst_p)
    y3 = _tc_sage_combine_scale(p2, y2, degi, dego, b2, W3)
    p3 = _sc_agg_run(y3, src_p, dst_p)
    return _tc_final(p3, degi, b3)
